# baseline (speedup 1.0000x reference)
import functools

import jax
import jax.numpy as jnp
from jax import lax
from jax.experimental import pallas as pl
from jax.experimental.pallas import tpu as pltpu
from jax.experimental.pallas import tpu_sc as plsc

_NUM_CORES = 2
_NUM_SUBCORES = 16
_NW = _NUM_CORES * _NUM_SUBCORES
_L = 16


@functools.lru_cache(maxsize=None)
def _make_sc_call(B, U, K, G):
    assert B % (_NW * _L) == 0 and K % _L == 0
    PB = B // _NW
    KC = K // _L

    mesh = plsc.VectorSubcoreMesh(
        core_axis_name="c", subcore_axis_name="s",
        num_cores=_NUM_CORES, num_subcores=_NUM_SUBCORES)

    def body(rows_hbm, cols_hbm, user_hbm, goods_hbm, out_hbm,
             rows_v, cols_v, gidx_v, u_v, g_v, out_v, sem_u, sem_g):
        wid = lax.axis_index("s") * _NUM_CORES + lax.axis_index("c")
        base = wid * PB

        pltpu.sync_copy(rows_hbm.at[pl.ds(base, PB)], rows_v)
        pltpu.sync_copy(cols_hbm.at[pl.ds(base, PB)], cols_v)

        cp_u = pltpu.make_async_copy(user_hbm.at[rows_v], u_v, sem_u)
        cp_u.start()

        lanes = lax.iota(jnp.int32, _L)
        koffs = [lanes * G + (_L * j * G) for j in range(KC)]

        def build(i, _):
            off = i * _L
            c16 = cols_v[pl.ds(off, _L)]
            for pp in range(_L):
                c = c16[pp]
                for j in range(KC):
                    gidx_v[pl.ds((off + pp) * K + _L * j, _L)] = koffs[j] + c
            return 0
        lax.fori_loop(0, PB // _L, build, 0)

        cps = [pltpu.make_async_copy(
                   goods_hbm.at[gidx_v.at[pl.ds(t * 128, 128)]],
                   g_v.at[pl.ds(t * 128, 128)], sem_g)
               for t in range(PB * K // 128)]
        for cp in cps:
            cp.start()
        cp_u.wait()
        for cp in cps:
            cp.wait()

        def comp(i, _):
            off = i * _L
            acc = jnp.zeros((_L,), jnp.float32)
            for pp in range(_L):
                p = off + pp
                s = jnp.zeros((_L,), jnp.float32)
                for j in range(KC):
                    s = s + (u_v[p, pl.ds(_L * j, _L)]
                             * g_v[pl.ds(p * K + _L * j, _L)])
                acc = jnp.where(lanes == pp, jnp.sum(s), acc)
            out_v[pl.ds(off, _L)] = acc
            return 0
        lax.fori_loop(0, PB // _L, comp, 0)

        pltpu.sync_copy(out_v, out_hbm.at[pl.ds(base, PB)])

    return pl.kernel(
        body,
        out_type=jax.ShapeDtypeStruct((B,), jnp.float32),
        mesh=mesh,
        scratch_types=[
            pltpu.VMEM((PB,), jnp.int32),
            pltpu.VMEM((PB,), jnp.int32),
            pltpu.VMEM((PB * K,), jnp.int32),
            pltpu.VMEM((PB, K), jnp.float32),
            pltpu.VMEM((PB * K,), jnp.float32),
            pltpu.VMEM((PB,), jnp.float32),
            pltpu.SemaphoreType.DMA,
            pltpu.SemaphoreType.DMA,
        ],
        compiler_params=pltpu.CompilerParams(
            needs_layout_passes=False, use_tc_tiling_on_sc=False),
        name="als_pair_dot_sc",
    )


def kernel(location, user_matrix, goods_matrix):
    B = location.shape[0]
    U, K = user_matrix.shape
    _, G = goods_matrix.shape
    rows = location[:, 0].astype(jnp.int32)
    cols = location[:, 1].astype(jnp.int32)
    goods_flat = goods_matrix.reshape(K * G)
    out = _make_sc_call(B, U, K, G)(rows, cols, user_matrix, goods_flat)
    return out[:, None]
